# baseline (speedup 1.0000x reference)
import functools

import jax
import jax.numpy as jnp
from jax import lax
from jax.experimental import pallas as pl
from jax.experimental.pallas import tpu as pltpu
from jax.experimental.pallas import tpu_sc as plsc

_N = 16384
_C = 1000
_NW = 32
_RPW = 32
_LANES = 16
_SCALE = 1.0 / (224.0 * 224.0)

_sc_mesh = plsc.VectorSubcoreMesh(core_axis_name="c", subcore_axis_name="s")


@functools.partial(
    pl.kernel,
    mesh=_sc_mesh,
    out_type=[
        jax.ShapeDtypeStruct((_NW, _RPW * _LANES), jnp.float32),
        jax.ShapeDtypeStruct((_NW, _RPW * _LANES), jnp.float32),
    ],
    scratch_types=[
        pltpu.VMEM((_N,), jnp.float32),
        pltpu.VMEM((_N,), jnp.float32),
        pltpu.VMEM((_RPW * _LANES,), jnp.float32),
        pltpu.VMEM((_RPW * _LANES,), jnp.float32),
    ],
)
def _sc_partial(x_hbm, t_hbm, sums_hbm, cnts_hbm, xbuf, tbuf, rsum, rcnt):
    cid = lax.axis_index("c")
    sid = lax.axis_index("s")
    wid = sid * 2 + cid

    zero16 = jnp.zeros((_LANES,), jnp.float32)

    def row_body(k, _):
        r = wid + _NW * k

        @pl.when(r < _C)
        def _do():
            pltpu.sync_copy(x_hbm.at[r], xbuf)
            pltpu.sync_copy(t_hbm.at[r], tbuf)

            def inner(j, carry):
                acc, cnt = carry
                base = j * (4 * _LANES)
                for u in range(4):
                    xv = xbuf[pl.ds(base + u * _LANES, _LANES)]
                    tv = tbuf[pl.ds(base + u * _LANES, _LANES)]
                    d = xv - tv
                    m = tv > 0.0
                    mf = jnp.where(m, 1.0, 0.0)
                    acc = acc + d * d * mf
                    cnt = cnt + mf
                return acc, cnt

            acc, cnt = lax.fori_loop(
                0, _N // (4 * _LANES), inner, (zero16, zero16))
            rsum[pl.ds(k * _LANES, _LANES)] = acc
            rcnt[pl.ds(k * _LANES, _LANES)] = cnt

        @pl.when(r >= _C)
        def _pad():
            rsum[pl.ds(k * _LANES, _LANES)] = zero16
            rcnt[pl.ds(k * _LANES, _LANES)] = zero16

        return 0

    lax.fori_loop(0, _RPW, row_body, 0)
    pltpu.sync_copy(rsum, sums_hbm.at[wid])
    pltpu.sync_copy(rcnt, cnts_hbm.at[wid])


def _combine_body(sums_ref, cnts_ref, out_ref):
    g = lax.broadcasted_iota(jnp.int32, (_RPW * _LANES, _RPW), 0) // _LANES
    h = lax.broadcasted_iota(jnp.int32, (_RPW * _LANES, _RPW), 1)
    sel = (g == h).astype(jnp.float32)
    s = jnp.dot(sums_ref[...], sel, preferred_element_type=jnp.float32)
    c = jnp.dot(cnts_ref[...], sel, preferred_element_type=jnp.float32)
    mean = jnp.where(c > 0.0, s / jnp.maximum(c, 1.0), 0.0)
    out_ref[0, 0] = jnp.sum(mean) * _SCALE


def kernel(inputs, targets):
    x_t = inputs.T
    t_t = targets.T
    sums, cnts = _sc_partial(x_t, t_t)
    out = pl.pallas_call(
        _combine_body,
        out_specs=pl.BlockSpec(memory_space=pltpu.SMEM),
        out_shape=jax.ShapeDtypeStruct((1, 1), jnp.float32),
    )(sums, cnts)
    return out[0, 0]


# speedup vs baseline: 2.5039x; 2.5039x over previous
import functools

import jax
import jax.numpy as jnp
from jax import lax
from jax.experimental import pallas as pl
from jax.experimental.pallas import tpu as pltpu
from jax.experimental.pallas import tpu_sc as plsc

_N = 16384
_C = 1000
_C_TC = 808
_SC_BASE = _C_TC
_NW = 32
_RPW = (_C - _C_TC) // _NW
_LANES = 16
_BLOCK_N = 2048
_SCALE = 1.0 / (224.0 * 224.0)


def _tc_body(x_ref, t_ref, out_ref, acc_sum, acc_cnt):
    i = pl.program_id(0)
    first = i == 0

    def strip(r, carry):
        rows = pl.ds(r * 8, 8)

        def tile(j):
            xj = x_ref[rows, j * 128:(j + 1) * 128]
            tj = t_ref[rows, j * 128:(j + 1) * 128]
            dj = xj - tj
            mj = tj > 0.0
            return jnp.where(mj, dj * dj, 0.0), mj.astype(jnp.float32)

        ps, pc = tile(0)
        for j in range(1, _BLOCK_N // 128):
            vj, cj = tile(j)
            ps = ps + vj
            pc = pc + cj
        prev_s = jnp.where(first, 0.0, acc_sum[rows, :])
        prev_c = jnp.where(first, 0.0, acc_cnt[rows, :])
        acc_sum[rows, :] = prev_s + ps
        acc_cnt[rows, :] = prev_c + pc
        return carry

    lax.fori_loop(0, _C_TC // 8, strip, 0, unroll=False)

    @pl.when(i == pl.num_programs(0) - 1)
    def _fin():
        s = jnp.sum(acc_sum[...], axis=1, keepdims=True)
        n = jnp.sum(acc_cnt[...], axis=1, keepdims=True)
        mean = jnp.where(n > 0.0, s / jnp.maximum(n, 1.0), 0.0)
        out_ref[0, 0] = jnp.sum(mean) * _SCALE


_sc_mesh = plsc.VectorSubcoreMesh(core_axis_name="c", subcore_axis_name="s")


@functools.partial(
    pl.kernel,
    mesh=_sc_mesh,
    out_type=[
        jax.ShapeDtypeStruct((_NW, _RPW * _LANES), jnp.float32),
        jax.ShapeDtypeStruct((_NW, _RPW * _LANES), jnp.float32),
    ],
    scratch_types=[
        pltpu.VMEM((_N,), jnp.float32),
        pltpu.VMEM((_N,), jnp.float32),
        pltpu.VMEM((_RPW * _LANES,), jnp.float32),
        pltpu.VMEM((_RPW * _LANES,), jnp.float32),
    ],
)
def _sc_partial(x_hbm, t_hbm, sums_hbm, cnts_hbm, xbuf, tbuf, rsum, rcnt):
    cid = lax.axis_index("c")
    sid = lax.axis_index("s")
    wid = sid * 2 + cid

    zero16 = jnp.zeros((_LANES,), jnp.float32)

    def row_body(k, _):
        r = _SC_BASE + wid + _NW * k
        pltpu.sync_copy(x_hbm.at[r], xbuf)
        pltpu.sync_copy(t_hbm.at[r], tbuf)

        def inner(j, carry):
            accs = list(carry)
            base = j * (8 * _LANES)
            for u in range(8):
                xv = xbuf[pl.ds(base + u * _LANES, _LANES)]
                tv = tbuf[pl.ds(base + u * _LANES, _LANES)]
                d = xv - tv
                m = tv > 0.0
                mf = jnp.where(m, 1.0, 0.0)
                accs[2 * (u % 4)] = accs[2 * (u % 4)] + d * d * mf
                accs[2 * (u % 4) + 1] = accs[2 * (u % 4) + 1] + mf
            return tuple(accs)

        carry = lax.fori_loop(0, _N // (8 * _LANES), inner, (zero16,) * 8)
        acc = carry[0] + carry[2] + carry[4] + carry[6]
        cnt = carry[1] + carry[3] + carry[5] + carry[7]
        rsum[pl.ds(k * _LANES, _LANES)] = acc
        rcnt[pl.ds(k * _LANES, _LANES)] = cnt
        return 0

    lax.fori_loop(0, _RPW, row_body, 0)
    pltpu.sync_copy(rsum, sums_hbm.at[wid])
    pltpu.sync_copy(rcnt, cnts_hbm.at[wid])


def _combine_body(sums_ref, cnts_ref, tc_ref, out_ref):
    g = lax.broadcasted_iota(jnp.int32, (_RPW * _LANES, _RPW), 0) // _LANES
    h = lax.broadcasted_iota(jnp.int32, (_RPW * _LANES, _RPW), 1)
    sel = (g == h).astype(jnp.float32)
    s = jnp.dot(sums_ref[...], sel, preferred_element_type=jnp.float32)
    c = jnp.dot(cnts_ref[...], sel, preferred_element_type=jnp.float32)
    mean = jnp.where(c > 0.0, s / jnp.maximum(c, 1.0), 0.0)
    out_ref[0, 0] = jnp.sum(mean) * _SCALE + tc_ref[0, 0]


def kernel(inputs, targets):
    x_t = inputs.T
    t_t = targets.T

    tc_part = pl.pallas_call(
        _tc_body,
        grid=(_N // _BLOCK_N,),
        in_specs=[
            pl.BlockSpec((_C_TC, _BLOCK_N), lambda i: (0, i)),
            pl.BlockSpec((_C_TC, _BLOCK_N), lambda i: (0, i)),
        ],
        out_specs=pl.BlockSpec(memory_space=pltpu.SMEM),
        out_shape=jax.ShapeDtypeStruct((1, 1), jnp.float32),
        scratch_shapes=[
            pltpu.VMEM((_C_TC, 128), jnp.float32),
            pltpu.VMEM((_C_TC, 128), jnp.float32),
        ],
    )(x_t, t_t)

    sums, cnts = _sc_partial(x_t, t_t)

    out = pl.pallas_call(
        _combine_body,
        in_specs=[
            pl.BlockSpec(memory_space=pltpu.VMEM),
            pl.BlockSpec(memory_space=pltpu.VMEM),
            pl.BlockSpec(memory_space=pltpu.SMEM),
        ],
        out_specs=pl.BlockSpec(memory_space=pltpu.SMEM),
        out_shape=jax.ShapeDtypeStruct((1, 1), jnp.float32),
    )(sums, cnts, tc_part)
    return out[0, 0]


# speedup vs baseline: 3.3377x; 1.3330x over previous
import jax
import jax.numpy as jnp
from jax.experimental import pallas as pl
from jax.experimental.pallas import tpu as pltpu

_N = 16384
_C = 1000
_BLOCK_N = 4096
_SCALE = 1.0 / (224.0 * 224.0)


def _body(x_ref, t_ref, out_ref, acc_sum, acc_cnt):
    i = pl.program_id(0)

    first = i == 0

    def strip(r, carry):
        rows = pl.ds(r * 8, 8)

        def tile(j):
            xj = x_ref[rows, j * 128:(j + 1) * 128]
            tj = t_ref[rows, j * 128:(j + 1) * 128]
            dj = xj - tj
            mj = tj > 0.0
            return jnp.where(mj, dj * dj, 0.0), mj.astype(jnp.float32)

        ps, pc = tile(0)
        for j in range(1, _BLOCK_N // 128):
            vj, cj = tile(j)
            ps = ps + vj
            pc = pc + cj
        prev_s = jnp.where(first, 0.0, acc_sum[rows, :])
        prev_c = jnp.where(first, 0.0, acc_cnt[rows, :])
        acc_sum[rows, :] = prev_s + ps
        acc_cnt[rows, :] = prev_c + pc
        return carry

    jax.lax.fori_loop(0, _C // 8, strip, 0, unroll=False)

    @pl.when(i == pl.num_programs(0) - 1)
    def _fin():
        s = jnp.sum(acc_sum[...], axis=1, keepdims=True)
        n = jnp.sum(acc_cnt[...], axis=1, keepdims=True)
        mean = jnp.where(n > 0.0, s / jnp.maximum(n, 1.0), 0.0)
        out_ref[0, 0] = jnp.sum(mean) * _SCALE


def kernel(inputs, targets):
    x_t = inputs.T
    t_t = targets.T
    grid = (_N // _BLOCK_N,)
    out = pl.pallas_call(
        _body,
        grid=grid,
        in_specs=[
            pl.BlockSpec((_C, _BLOCK_N), lambda i: (0, i)),
            pl.BlockSpec((_C, _BLOCK_N), lambda i: (0, i)),
        ],
        out_specs=pl.BlockSpec(memory_space=pltpu.SMEM),
        out_shape=jax.ShapeDtypeStruct((1, 1), jnp.float32),
        compiler_params=pltpu.CompilerParams(
            vmem_limit_bytes=100 * 1024 * 1024,
        ),
        scratch_shapes=[
            pltpu.VMEM((_C, 128), jnp.float32),
            pltpu.VMEM((_C, 128), jnp.float32),
        ],
    )(x_t, t_t)
    return out[0, 0]
